# Initial kernel scaffold; baseline (speedup 1.0000x reference)
import jax
import jax.numpy as jnp
from jax.experimental import pallas as pl


def kernel(Attention_map):
    raise NotImplementedError("write your pallas kernel here")


# speedup vs baseline: 13.2566x; 13.2566x over previous
import functools

import numpy as np
import jax
import jax.numpy as jnp
from jax import lax
from jax.experimental import pallas as pl
from jax.experimental.pallas import tpu as pltpu
from jax.experimental.pallas import tpu_sc as plsc

_TOPK = 0.2
_LANES = 16
_UNROLL = 4


def _orderable(v):
    b = plsc.bitcast(v, jnp.int32)
    s = lax.shift_right_arithmetic(b, 31)
    return lax.bitwise_xor(b, lax.bitwise_and(s, jnp.int32(0x7FFFFFFF)))


def _make_sc_kernel(n_rows, n_cols, k):
    info = plsc.get_sparse_core_info()
    nc, ns = info.num_cores, info.num_subcores
    n_workers = nc * ns
    assert n_rows % n_workers == 0
    rows_per_w = n_rows // n_workers
    n_vregs = n_cols // _LANES
    assert n_vregs % _UNROLL == 0

    mesh = plsc.VectorSubcoreMesh(core_axis_name="c", subcore_axis_name="s")

    @functools.partial(
        pl.kernel,
        out_type=jax.ShapeDtypeStruct((n_rows, n_cols), jnp.float32),
        mesh=mesh,
        compiler_params=pltpu.CompilerParams(needs_layout_passes=False),
        scratch_types=[
            pltpu.VMEM((n_cols,), jnp.float32),
            pltpu.VMEM((256 * _LANES,), jnp.int32),
        ],
    )
    def sc_kernel(x_hbm, out_hbm, xbuf, bins):
        wid = lax.axis_index("s") * nc + lax.axis_index("c")
        lanes = lax.broadcasted_iota(jnp.int32, (_LANES,), 0)
        ones = jnp.ones((_LANES,), jnp.int32)

        def zero_bins():
            def zb(i, c):
                for u in range(8):
                    bins[pl.ds((i * 8 + u) * _LANES, _LANES)] = jnp.zeros(
                        (_LANES,), jnp.int32)
                return c
            lax.fori_loop(0, 256 // 8, zb, 0)

        def binsum(b):
            return jnp.sum(bins[pl.ds(b * _LANES, _LANES)])

        def pick_bin(k_cur):
            def cond(c):
                b, acc = c
                return acc + binsum(b) < k_cur

            def body(c):
                b, acc = c
                return b - 1, acc + binsum(b)

            b_sel, acc = lax.while_loop(
                cond, body, (jnp.int32(255), jnp.int32(0)))
            return b_sel, k_cur - acc

        def hist_scan(level, prefix):
            zero_bins()
            shift_byte = 24 - 8 * level

            def body(i, c):
                for uu in range(_UNROLL):
                    off = (i * _UNROLL + uu) * _LANES
                    u = _orderable(xbuf[pl.ds(off, _LANES)])
                    if level == 0:
                        byte = lax.shift_right_arithmetic(u, 24) + 128
                        idx = lax.bitwise_or(lax.shift_left(byte, 4), lanes)
                        plsc.addupdate_scatter(bins, [idx], ones)
                    else:
                        shift_match = 32 - 8 * level
                        m = lax.shift_right_arithmetic(u, shift_match) == prefix
                        byte = lax.bitwise_and(
                            lax.shift_right_arithmetic(u, shift_byte),
                            jnp.int32(0xFF))
                        idx = lax.bitwise_or(lax.shift_left(byte, 4), lanes)
                        plsc.addupdate_scatter(bins, [idx], ones, mask=m)
                return c

            lax.fori_loop(0, n_vregs // _UNROLL, body, 0)

        def per_row(r, c):
            row = wid * rows_per_w + r
            pltpu.sync_copy(x_hbm.at[row], xbuf)

            hist_scan(0, None)
            b0, k1 = pick_bin(jnp.int32(k))
            p = b0 - 128
            hist_scan(1, p)
            b1, k2 = pick_bin(k1)
            p = lax.bitwise_or(lax.shift_left(p, 8), b1)
            hist_scan(2, p)
            b2, k3 = pick_bin(k2)
            p = lax.bitwise_or(lax.shift_left(p, 8), b2)
            hist_scan(3, p)
            b3, _ = pick_bin(k3)
            ut = lax.bitwise_or(lax.shift_left(p, 8), b3)

            def flip(i, cc):
                for uu in range(_UNROLL):
                    off = (i * _UNROLL + uu) * _LANES
                    v = xbuf[pl.ds(off, _LANES)]
                    u = _orderable(v)
                    y = jnp.where(u >= ut, jnp.float32(1.0) - v, v)
                    xbuf[pl.ds(off, _LANES)] = y
                return cc

            lax.fori_loop(0, n_vregs // _UNROLL, flip, 0)
            pltpu.sync_copy(xbuf, out_hbm.at[row])
            return c

        lax.fori_loop(0, rows_per_w, per_row, 0)

    return sc_kernel


def kernel(Attention_map):
    B, C, H, W = Attention_map.shape
    L = C * H * W
    K = int(np.clip(int(L * _TOPK), 1, C))
    x = Attention_map.reshape(B, L)
    sc = _make_sc_kernel(B, L, K)
    y = sc(x)
    return y.reshape(B, C, H, W)


# speedup vs baseline: 36.7044x; 2.7688x over previous
import functools

import numpy as np
import jax
import jax.numpy as jnp
from jax import lax
from jax.experimental import pallas as pl
from jax.experimental.pallas import tpu as pltpu
from jax.experimental.pallas import tpu_sc as plsc

_TOPK = 0.2
_LANES = 16
_UNROLL = 8


def _make_sc_kernel(n_rows, n_cols, k):
    info = plsc.get_sparse_core_info()
    nc, ns = info.num_cores, info.num_subcores
    n_workers = nc * ns
    assert n_rows % n_workers == 0
    rows_per_w = n_rows // n_workers
    assert (n_cols // _LANES) % _UNROLL == 0

    mesh = plsc.VectorSubcoreMesh(core_axis_name="c", subcore_axis_name="s")

    @functools.partial(
        pl.kernel,
        out_type=jax.ShapeDtypeStruct((n_rows, n_cols), jnp.float32),
        mesh=mesh,
        compiler_params=pltpu.CompilerParams(needs_layout_passes=False),
        scratch_types=[
            pltpu.VMEM((n_cols,), jnp.float32),
            pltpu.VMEM((n_cols,), jnp.float32),
            pltpu.VMEM((n_cols,), jnp.int32),
            pltpu.VMEM((256 * _LANES,), jnp.int32),
            pltpu.SemaphoreType.DMA,
            pltpu.SemaphoreType.DMA,
        ],
    )
    def sc_kernel(x_hbm, out_hbm, xb0, xb1, ubuf, bins, insem, outsem):
        wid = lax.axis_index("s") * nc + lax.axis_index("c")
        lanes = lax.broadcasted_iota(jnp.int32, (_LANES,), 0)
        ones = jnp.ones((_LANES,), jnp.int32)
        base_row = wid * rows_per_w

        def binsum(b):
            return jnp.sum(bins[pl.ds(b * _LANES, _LANES)])

        def pick_bin(k_cur):
            def cond(c):
                b, acc = c
                return acc + binsum(b) < k_cur

            def body(c):
                b, acc = c
                return b - 1, acc + binsum(b)

            b_sel, acc = lax.while_loop(
                cond, body, (jnp.int32(255), jnp.int32(0)))
            return b_sel, k_cur - acc

        def zero_bins():
            @plsc.parallel_loop(0, 256 * _LANES, _LANES, unroll=_UNROLL)
            def _(off):
                bins[pl.ds(off, _LANES)] = jnp.zeros((_LANES,), jnp.int32)

        bufs = [xb0, xb1]
        pltpu.async_copy(x_hbm.at[base_row], xb0, insem)

        for r in range(rows_per_w):
            row = bufs[r % 2]
            pltpu.make_async_copy(x_hbm.at[base_row + r], row, insem).wait()
            if r + 1 < rows_per_w:
                if r >= 1:
                    pltpu.make_async_copy(
                        bufs[(r + 1) % 2],
                        out_hbm.at[base_row + r - 1], outsem).wait()
                pltpu.async_copy(x_hbm.at[base_row + r + 1],
                                 bufs[(r + 1) % 2], insem)

            zero_bins()

            @plsc.parallel_loop(0, n_cols, _LANES, unroll=_UNROLL)
            def _(off):
                b = plsc.bitcast(row[pl.ds(off, _LANES)], jnp.int32)
                s = lax.shift_right_arithmetic(b, 31)
                u = lax.bitwise_xor(
                    b, lax.bitwise_and(s, jnp.int32(0x7FFFFFFF)))
                ubuf[pl.ds(off, _LANES)] = u
                byte = lax.shift_right_arithmetic(u, 24) + 128
                idx = lax.bitwise_or(lax.shift_left(byte, 4), lanes)
                plsc.addupdate_scatter(bins, [idx], ones)

            b0, k1 = pick_bin(jnp.int32(k))
            p1 = b0 - 128

            def masked_scan(prefix, shift_match, shift_byte):
                zero_bins()

                @plsc.parallel_loop(0, n_cols, _LANES, unroll=_UNROLL)
                def _(off):
                    u = ubuf[pl.ds(off, _LANES)]
                    m = lax.shift_right_arithmetic(u, shift_match) == prefix
                    byte = lax.bitwise_and(
                        lax.shift_right_arithmetic(u, shift_byte),
                        jnp.int32(0xFF))
                    idx = lax.bitwise_or(lax.shift_left(byte, 4), lanes)
                    plsc.addupdate_scatter(bins, [idx], ones, mask=m)

            masked_scan(p1, 24, 16)
            b1, k2 = pick_bin(k1)
            p2 = lax.bitwise_or(lax.shift_left(p1, 8), b1)
            masked_scan(p2, 16, 8)
            b2, k3 = pick_bin(k2)
            p3 = lax.bitwise_or(lax.shift_left(p2, 8), b2)
            masked_scan(p3, 8, 0)
            b3, _ = pick_bin(k3)
            ut = lax.bitwise_or(lax.shift_left(p3, 8), b3)

            @plsc.parallel_loop(0, n_cols, _LANES, unroll=_UNROLL)
            def _(off):
                v = row[pl.ds(off, _LANES)]
                u = ubuf[pl.ds(off, _LANES)]
                y = jnp.where(u >= ut, jnp.float32(1.0) - v, v)
                row[pl.ds(off, _LANES)] = y

            pltpu.async_copy(row, out_hbm.at[base_row + r], outsem)

        pltpu.make_async_copy(
            bufs[(rows_per_w - 1) % 2],
            out_hbm.at[base_row + rows_per_w - 1], outsem).wait()
        pltpu.make_async_copy(
            bufs[rows_per_w % 2],
            out_hbm.at[base_row + rows_per_w - 2], outsem).wait()

    return sc_kernel


def kernel(Attention_map):
    B, C, H, W = Attention_map.shape
    L = C * H * W
    K = int(np.clip(int(L * _TOPK), 1, C))
    x = Attention_map.reshape(B, L)
    sc = _make_sc_kernel(B, L, K)
    y = sc(x)
    return y.reshape(B, C, H, W)
